# Initial kernel scaffold; baseline (speedup 1.0000x reference)
import jax
import jax.numpy as jnp
from jax.experimental import pallas as pl


def kernel(x, W1, b1, W2, b2, top_k):
    raise NotImplementedError("write your pallas kernel here")


# speedup vs baseline: 5.2837x; 5.2837x over previous
import jax
import jax.numpy as jnp
from jax.experimental import pallas as pl

_BB = 512
_K = 8


def _gate_body(x_ref, w1_ref, b1_ref, w2_ref, b2_ref, tk_ref, o_ref):
    h = jnp.dot(x_ref[...], w1_ref[...], preferred_element_type=jnp.float32)
    h = jnp.maximum(h + b1_ref[...], 0.0)
    logits = jnp.dot(h, w2_ref[...], preferred_element_type=jnp.float32)
    logits = logits + b2_ref[...]
    m = jnp.max(logits, axis=-1, keepdims=True)
    e = jnp.exp(logits - m)
    w = e / jnp.sum(e, axis=-1, keepdims=True)
    bb, E = w.shape
    iota = jax.lax.broadcasted_iota(jnp.int32, (bb, E), 1)
    cur = w
    selected = jnp.zeros((bb, E), jnp.bool_)
    for _ in range(_K):
        mx = jnp.max(cur, axis=-1, keepdims=True)
        is_mx = cur == mx
        idxm = jnp.where(is_mx, iota, E)
        mn = jnp.min(idxm, axis=-1, keepdims=True)
        first = iota == mn
        selected = selected | first
        cur = jnp.where(first, -1.0, cur)
    sp = jnp.where(selected, w, 0.0)
    s = jnp.sum(sp, axis=-1, keepdims=True)
    sp = sp / (s + 1e-8)
    tk = tk_ref[0, 0]
    use_sparse = jnp.logical_and(tk > 0, tk < E)
    o_ref[...] = jnp.where(use_sparse, sp, w)


def kernel(x, W1, b1, W2, b2, top_k):
    B, D = x.shape
    H = W1.shape[1]
    E = W2.shape[1]
    b1r = b1.reshape(1, H)
    b2r = b2.reshape(1, E)
    tk = jnp.asarray(top_k, jnp.int32).reshape(1, 1)
    grid = (B // _BB,)
    return pl.pallas_call(
        _gate_body,
        grid=grid,
        in_specs=[
            pl.BlockSpec((_BB, D), lambda i: (i, 0)),
            pl.BlockSpec((D, H), lambda i: (0, 0)),
            pl.BlockSpec((1, H), lambda i: (0, 0)),
            pl.BlockSpec((H, E), lambda i: (0, 0)),
            pl.BlockSpec((1, E), lambda i: (0, 0)),
            pl.BlockSpec((1, 1), lambda i: (0, 0)),
        ],
        out_specs=pl.BlockSpec((_BB, E), lambda i: (i, 0)),
        out_shape=jax.ShapeDtypeStruct((B, E), jnp.float32),
    )(x, W1, b1r, W2, b2r, tk)


# speedup vs baseline: 7.9564x; 1.5058x over previous
import jax
import jax.numpy as jnp
from jax.experimental import pallas as pl

_BB = 512
_K = 8


def _gate_body(x_ref, w1_ref, b1_ref, w2_ref, b2_ref, tk_ref, o_ref):
    h = jnp.dot(x_ref[...], w1_ref[...], preferred_element_type=jnp.float32)
    h = jnp.maximum(h + b1_ref[...], 0.0)
    logits = jnp.dot(h, w2_ref[...], preferred_element_type=jnp.float32)
    logits = logits + b2_ref[...]
    E = logits.shape[-1]
    m = jnp.max(logits, axis=-1, keepdims=True)
    e = jnp.exp(logits - m)
    z = jnp.sum(e, axis=-1, keepdims=True)
    cur = e
    for _ in range(_K):
        t = jnp.max(cur, axis=-1, keepdims=True)
        cur = jnp.where(cur >= t, -1.0, cur)
    sel = e >= t
    se = jnp.sum(jnp.where(sel, e, 0.0), axis=-1, keepdims=True)
    tk = tk_ref[0, 0]
    use_sparse = jnp.logical_and(tk > 0, tk < E)
    num = jnp.where(jnp.logical_or(sel, jnp.logical_not(use_sparse)), e, 0.0)
    den = jnp.where(use_sparse, se + 1e-8 * z, z)
    o_ref[...] = num / den


def kernel(x, W1, b1, W2, b2, top_k):
    B, D = x.shape
    H = W1.shape[1]
    E = W2.shape[1]
    b1r = b1.reshape(1, H)
    b2r = b2.reshape(1, E)
    tk = jnp.asarray(top_k, jnp.int32).reshape(1, 1)
    grid = (B // _BB,)
    return pl.pallas_call(
        _gate_body,
        grid=grid,
        in_specs=[
            pl.BlockSpec((_BB, D), lambda i: (i, 0)),
            pl.BlockSpec((D, H), lambda i: (0, 0)),
            pl.BlockSpec((1, H), lambda i: (0, 0)),
            pl.BlockSpec((H, E), lambda i: (0, 0)),
            pl.BlockSpec((1, E), lambda i: (0, 0)),
            pl.BlockSpec((1, 1), lambda i: (0, 0)),
        ],
        out_specs=pl.BlockSpec((_BB, E), lambda i: (i, 0)),
        out_shape=jax.ShapeDtypeStruct((B, E), jnp.float32),
    )(x, W1, b1r, W2, b2r, tk)


# speedup vs baseline: 10.0562x; 1.2639x over previous
import jax
import jax.numpy as jnp
from jax.experimental import pallas as pl

_BB = 2048
_K = 8


def _gate_body(x_ref, w1_ref, b1_ref, w2_ref, b2_ref, tk_ref, o_ref):
    h = jnp.dot(x_ref[...], w1_ref[...], preferred_element_type=jnp.float32)
    h = jnp.maximum(h + b1_ref[...], 0.0)
    logits = jnp.dot(h, w2_ref[...], preferred_element_type=jnp.float32)
    logits = logits + b2_ref[...]
    E = logits.shape[-1]
    m = jnp.max(logits, axis=-1, keepdims=True)
    e = jnp.exp(logits - m)
    z = jnp.sum(e, axis=-1, keepdims=True)
    cur = e
    for _ in range(_K):
        t = jnp.max(cur, axis=-1, keepdims=True)
        cur = jnp.where(cur >= t, -1.0, cur)
    sel = e >= t
    se = jnp.sum(jnp.where(sel, e, 0.0), axis=-1, keepdims=True)
    tk = tk_ref[0, 0]
    use_sparse = jnp.logical_and(tk > 0, tk < E)
    num = jnp.where(jnp.logical_or(sel, jnp.logical_not(use_sparse)), e, 0.0)
    den = jnp.where(use_sparse, se + 1e-8 * z, z)
    o_ref[...] = num / den


def kernel(x, W1, b1, W2, b2, top_k):
    B, D = x.shape
    H = W1.shape[1]
    E = W2.shape[1]
    b1r = b1.reshape(1, H)
    b2r = b2.reshape(1, E)
    tk = jnp.asarray(top_k, jnp.int32).reshape(1, 1)
    grid = (B // _BB,)
    return pl.pallas_call(
        _gate_body,
        grid=grid,
        in_specs=[
            pl.BlockSpec((_BB, D), lambda i: (i, 0)),
            pl.BlockSpec((D, H), lambda i: (0, 0)),
            pl.BlockSpec((1, H), lambda i: (0, 0)),
            pl.BlockSpec((H, E), lambda i: (0, 0)),
            pl.BlockSpec((1, E), lambda i: (0, 0)),
            pl.BlockSpec((1, 1), lambda i: (0, 0)),
        ],
        out_specs=pl.BlockSpec((_BB, E), lambda i: (i, 0)),
        out_shape=jax.ShapeDtypeStruct((B, E), jnp.float32),
    )(x, W1, b1r, W2, b2r, tk)


# speedup vs baseline: 10.2643x; 1.0207x over previous
import jax
import jax.numpy as jnp
from jax.experimental import pallas as pl

_BB = 2048
_K = 8


def _gate_body(x_ref, w1_ref, b1_ref, w2_ref, b2_ref, tk_ref, o_ref):
    h = jnp.dot(x_ref[...], w1_ref[...], preferred_element_type=jnp.float32)
    h = jnp.maximum(h + b1_ref[...], 0.0)
    logits = jnp.dot(h, w2_ref[...], preferred_element_type=jnp.float32)
    logits = logits + b2_ref[...]
    E = logits.shape[-1]
    m = jnp.max(logits, axis=-1, keepdims=True)
    cur = jnp.where(logits >= m, -1e30, logits)
    for _ in range(_K - 2):
        t = jnp.max(cur, axis=-1, keepdims=True)
        cur = jnp.where(cur >= t, -1e30, cur)
    t = jnp.max(cur, axis=-1, keepdims=True)
    sel = logits >= t
    e = jnp.exp(logits - m)
    z = jnp.sum(e, axis=-1, keepdims=True)
    se = jnp.sum(jnp.where(sel, e, 0.0), axis=-1, keepdims=True)
    tk = tk_ref[0, 0]
    use_sparse = jnp.logical_and(tk > 0, tk < E)
    num = jnp.where(jnp.logical_or(sel, jnp.logical_not(use_sparse)), e, 0.0)
    den = jnp.where(use_sparse, se + 1e-8 * z, z)
    o_ref[...] = num / den


def kernel(x, W1, b1, W2, b2, top_k):
    B, D = x.shape
    H = W1.shape[1]
    E = W2.shape[1]
    b1r = b1.reshape(1, H)
    b2r = b2.reshape(1, E)
    tk = jnp.asarray(top_k, jnp.int32).reshape(1, 1)
    grid = (B // _BB,)
    return pl.pallas_call(
        _gate_body,
        grid=grid,
        in_specs=[
            pl.BlockSpec((_BB, D), lambda i: (i, 0)),
            pl.BlockSpec((D, H), lambda i: (0, 0)),
            pl.BlockSpec((1, H), lambda i: (0, 0)),
            pl.BlockSpec((H, E), lambda i: (0, 0)),
            pl.BlockSpec((1, E), lambda i: (0, 0)),
            pl.BlockSpec((1, 1), lambda i: (0, 0)),
        ],
        out_specs=pl.BlockSpec((_BB, E), lambda i: (i, 0)),
        out_shape=jax.ShapeDtypeStruct((B, E), jnp.float32),
    )(x, W1, b1r, W2, b2r, tk)
